# Initial kernel scaffold; baseline (speedup 1.0000x reference)
import jax
import jax.numpy as jnp
from jax.experimental import pallas as pl


def kernel(token_ids, table):
    raise NotImplementedError("write your pallas kernel here")


# speedup vs baseline: 2.9776x; 2.9776x over previous
import functools

import jax
import jax.numpy as jnp
from jax import lax
from jax.experimental import pallas as pl
from jax.experimental.pallas import tpu as pltpu
from jax.experimental.pallas import tpu_sc as plsc

D_MODEL = 128
CHUNK = 128


@functools.lru_cache(maxsize=None)
def _make_kernel(b_flat: int, vocab: int):
    info = plsc.get_sparse_core_info()
    nc, ns = info.num_cores, info.num_subcores
    nw = nc * ns
    b_per_w = b_flat // nw
    n_chunks = b_per_w // CHUNK
    mesh = plsc.VectorSubcoreMesh(core_axis_name="c", subcore_axis_name="s")

    @functools.partial(
        pl.kernel,
        mesh=mesh,
        out_type=jax.ShapeDtypeStruct((b_flat, D_MODEL), jnp.float32),
        scratch_types=[
            pltpu.VMEM((n_chunks, CHUNK), jnp.int32),
            pltpu.VMEM((CHUNK, D_MODEL), jnp.float32),
            pltpu.SemaphoreType.DMA,
        ],
    )
    def gather_kernel(idx_hbm, table_hbm, out_hbm, idx_v, rows_v, sem):
        wid = lax.axis_index("s") * nc + lax.axis_index("c")
        pltpu.sync_copy(idx_hbm.at[wid], idx_v)
        base = wid * b_per_w

        def body(j, carry):
            pltpu.async_copy(table_hbm.at[idx_v.at[j]], rows_v, sem).wait()
            pltpu.sync_copy(rows_v, out_hbm.at[pl.ds(base + j * CHUNK, CHUNK)])
            return carry

        lax.fori_loop(0, n_chunks, body, 0)

    return gather_kernel


def kernel(token_ids, table):
    b, s = token_ids.shape
    vocab, d = table.shape
    b_flat = b * s
    info = plsc.get_sparse_core_info()
    nw = info.num_cores * info.num_subcores
    b_per_w = b_flat // nw
    idx = token_ids.reshape(nw, b_per_w // CHUNK, CHUNK).astype(jnp.int32)
    out = _make_kernel(b_flat, vocab)(idx, table)
    return out.reshape(b, s, d)


# speedup vs baseline: 3.3353x; 1.1201x over previous
import functools

import jax
import jax.numpy as jnp
from jax import lax
from jax.experimental import pallas as pl
from jax.experimental.pallas import tpu as pltpu
from jax.experimental.pallas import tpu_sc as plsc

D_MODEL = 128
CHUNK = 128
NBUF = 4


@functools.lru_cache(maxsize=None)
def _make_kernel(b_flat: int, vocab: int):
    info = plsc.get_sparse_core_info()
    nc, ns = info.num_cores, info.num_subcores
    nw = nc * ns
    b_per_w = b_flat // nw
    n_chunks = b_per_w // CHUNK
    n_groups = (n_chunks + NBUF - 1) // NBUF
    mesh = plsc.VectorSubcoreMesh(core_axis_name="c", subcore_axis_name="s")

    @functools.partial(
        pl.kernel,
        mesh=mesh,
        out_type=jax.ShapeDtypeStruct((b_flat, D_MODEL), jnp.float32),
        scratch_types=(
            [pltpu.VMEM((n_chunks, CHUNK), jnp.int32)]
            + [pltpu.VMEM((CHUNK, D_MODEL), jnp.float32)] * NBUF
            + [pltpu.SemaphoreType.DMA] * (2 * NBUF)
        ),
    )
    def gather_kernel(idx_hbm, table_hbm, out_hbm, idx_v, *rest):
        bufs = rest[:NBUF]
        gsems = rest[NBUF : 2 * NBUF]
        wsems = rest[2 * NBUF :]

        wid = lax.axis_index("s") * nc + lax.axis_index("c")
        pltpu.sync_copy(idx_hbm.at[wid], idx_v)
        base = wid * b_per_w

        def start_gather(j, b):
            pltpu.async_copy(table_hbm.at[idx_v.at[j]], bufs[b], gsems[b])

        def wait_gather(b):
            pltpu.make_async_copy(
                table_hbm.at[pl.ds(0, CHUNK)], bufs[b], gsems[b]
            ).wait()

        def start_wb(j, b):
            pltpu.async_copy(
                bufs[b], out_hbm.at[pl.ds(base + j * CHUNK, CHUNK)], wsems[b]
            )

        def wait_wb(b):
            pltpu.make_async_copy(
                bufs[b], out_hbm.at[pl.ds(0, CHUNK)], wsems[b]
            ).wait()

        for b in range(NBUF):
            start_gather(b, b)

        def group(g, carry):
            for b in range(NBUF):
                j = NBUF * g + b

                @pl.when(j < n_chunks)
                def _():
                    wait_gather(b)
                    start_wb(j, b)

                    @pl.when(j + NBUF < n_chunks)
                    def _():
                        wait_wb(b)
                        start_gather(j + NBUF, b)

            return carry

        lax.fori_loop(0, n_groups, group, 0)

        for b in range(NBUF):
            wait_wb(b)

    return gather_kernel


def kernel(token_ids, table):
    b, s = token_ids.shape
    vocab, d = table.shape
    b_flat = b * s
    info = plsc.get_sparse_core_info()
    nw = info.num_cores * info.num_subcores
    b_per_w = b_flat // nw
    idx = token_ids.reshape(nw, b_per_w // CHUNK, CHUNK).astype(jnp.int32)
    out = _make_kernel(b_flat, vocab)(idx, table)
    return out.reshape(b, s, d)


# speedup vs baseline: 5.9707x; 1.7902x over previous
import functools

import jax
import jax.numpy as jnp
from jax import lax
from jax.experimental import pallas as pl
from jax.experimental.pallas import tpu as pltpu
from jax.experimental.pallas import tpu_sc as plsc

NBUF = 4


@functools.lru_cache(maxsize=None)
def _make_kernel(b: int, s: int, vocab: int, d: int):
    info = plsc.get_sparse_core_info()
    nc, ns = info.num_cores, info.num_subcores
    nw = nc * ns
    bes_per_w = b // nw
    n_groups = bes_per_w // NBUF
    mesh = plsc.VectorSubcoreMesh(core_axis_name="c", subcore_axis_name="s")

    @functools.partial(
        pl.kernel,
        mesh=mesh,
        out_type=jax.ShapeDtypeStruct((b, s, d), jnp.float32),
        scratch_types=(
            [pltpu.VMEM((bes_per_w, s), jnp.int32)]
            + [pltpu.VMEM((s, d), jnp.float32)] * NBUF
            + [pltpu.SemaphoreType.DMA] * (2 * NBUF)
        ),
    )
    def gather_kernel(idx_hbm, table_hbm, out_hbm, idx_v, *rest):
        bufs = rest[:NBUF]
        gsems = rest[NBUF : 2 * NBUF]
        wsems = rest[2 * NBUF :]

        wid = lax.axis_index("s") * nc + lax.axis_index("c")
        base = wid * bes_per_w
        pltpu.sync_copy(idx_hbm.at[pl.ds(base, bes_per_w)], idx_v)

        def start_gather(i, b_):
            pltpu.async_copy(table_hbm.at[idx_v.at[i]], bufs[b_], gsems[b_])

        def wait_gather(b_):
            pltpu.make_async_copy(out_hbm.at[0], bufs[b_], gsems[b_]).wait()

        def start_wb(i, b_):
            pltpu.async_copy(bufs[b_], out_hbm.at[base + i], wsems[b_])

        def wait_wb(b_):
            pltpu.make_async_copy(bufs[b_], out_hbm.at[0], wsems[b_]).wait()

        for b_ in range(NBUF):
            start_gather(b_, b_)

        def group(g, carry):
            for b_ in range(NBUF):
                i = NBUF * g + b_
                wait_gather(b_)
                start_wb(i, b_)

                @pl.when(i + NBUF < bes_per_w)
                def _():
                    wait_wb(b_)
                    start_gather(i + NBUF, b_)

            return carry

        lax.fori_loop(0, n_groups, group, 0)

        for b_ in range(NBUF):
            wait_wb(b_)

    return gather_kernel


def kernel(token_ids, table):
    b, s = token_ids.shape
    vocab, d = table.shape
    idx = jnp.asarray(token_ids, jnp.int32)
    return _make_kernel(b, s, vocab, d)(idx, table)


# speedup vs baseline: 10.4465x; 1.7496x over previous
import functools

import jax
import jax.numpy as jnp
from jax import lax
from jax.experimental import pallas as pl
from jax.experimental.pallas import tpu as pltpu
from jax.experimental.pallas import tpu_sc as plsc

D_MODEL = 128
CHUNK = 128
NBUF = 5


@functools.lru_cache(maxsize=None)
def _make_kernel(b_flat: int, vocab: int):
    info = plsc.get_sparse_core_info()
    nc, ns = info.num_cores, info.num_subcores
    nw = nc * ns
    b_per_w = b_flat // nw
    n_chunks = b_per_w // CHUNK
    n_groups = (n_chunks + NBUF - 1) // NBUF
    mesh = plsc.VectorSubcoreMesh(core_axis_name="c", subcore_axis_name="s")

    @functools.partial(
        pl.kernel,
        mesh=mesh,
        out_type=jax.ShapeDtypeStruct((b_flat, D_MODEL), jnp.float32),
        scratch_types=(
            [pltpu.VMEM((b_per_w,), jnp.int32)]
            + [pltpu.VMEM((CHUNK, D_MODEL), jnp.float32)] * NBUF
            + [pltpu.SemaphoreType.DMA] * (2 * NBUF)
        ),
    )
    def gather_kernel(idx_hbm, table_hbm, out_hbm, idx_v, *rest):
        bufs = rest[:NBUF]
        gsems = rest[NBUF : 2 * NBUF]
        wsems = rest[2 * NBUF :]

        wid = lax.axis_index("s") * nc + lax.axis_index("c")
        base = wid * b_per_w
        pltpu.sync_copy(idx_hbm.at[pl.ds(base, b_per_w)], idx_v)

        def start_gather(j, b_):
            pltpu.async_copy(
                table_hbm.at[idx_v.at[pl.ds(j * CHUNK, CHUNK)]], bufs[b_], gsems[b_]
            )

        def wait_gather(b_):
            pltpu.make_async_copy(
                out_hbm.at[pl.ds(0, CHUNK)], bufs[b_], gsems[b_]
            ).wait()

        def start_wb(j, b_):
            pltpu.async_copy(
                bufs[b_], out_hbm.at[pl.ds(base + j * CHUNK, CHUNK)], wsems[b_]
            )

        def wait_wb(b_):
            pltpu.make_async_copy(
                bufs[b_], out_hbm.at[pl.ds(0, CHUNK)], wsems[b_]
            ).wait()

        for b_ in range(NBUF):
            start_gather(b_, b_)

        def group(g, carry):
            for b_ in range(NBUF):
                j = NBUF * g + b_
                wait_gather(b_)
                start_wb(j, b_)

                @pl.when(j + NBUF < n_chunks)
                def _():
                    wait_wb(b_)
                    start_gather(j + NBUF, b_)

            return carry

        lax.fori_loop(0, n_groups, group, 0)

        for b_ in range(NBUF):
            wait_wb(b_)

    return gather_kernel


def kernel(token_ids, table):
    b, s = token_ids.shape
    vocab, d = table.shape
    b_flat = b * s
    idx = jnp.asarray(token_ids, jnp.int32).T.reshape(b_flat)
    out = _make_kernel(b_flat, vocab)(idx, table)
    return out.reshape(s, b, d).transpose(1, 0, 2)
